# Initial kernel scaffold; baseline (speedup 1.0000x reference)
import jax
import jax.numpy as jnp
from jax.experimental import pallas as pl


def kernel(x, input_ids, gate_w, W1, W2, W3, sw1, sw2, sw3):
    raise NotImplementedError("write your pallas kernel here")


# speedup vs baseline: 1.1010x; 1.1010x over previous
import functools

import jax
import jax.numpy as jnp
from jax.experimental import pallas as pl
from jax.experimental.pallas import tpu as pltpu

B, T, D = 1, 2048, 1024
E, K = 8, 2
INTER = 512
LIMIT = 10.0
NT = 8
TBLK = T // NT


def _moe_body(flat_ref, gate_ref, w1_ref, w2_ref, w3_ref, s1_ref, s2_ref, s3_ref,
              out_ref, wte_ref):
    e = pl.program_id(1)
    xb = flat_ref[...]

    @pl.when(e == 0)
    def _prologue():
        s = jnp.dot(xb, gate_ref[...], preferred_element_type=jnp.float32)
        scores = jnp.sqrt(jax.nn.softplus(s))
        iota = jax.lax.broadcasted_iota(jnp.int32, (TBLK, E), 1)
        m1 = jnp.max(scores, axis=1, keepdims=True)
        idx1 = jnp.min(jnp.where(scores == m1, iota, E), axis=1, keepdims=True)
        mask1 = iota == idx1
        scores2 = jnp.where(mask1, -jnp.inf, scores)
        m2 = jnp.max(scores2, axis=1, keepdims=True)
        idx2 = jnp.min(jnp.where(scores2 == m2, iota, E), axis=1, keepdims=True)
        mask2 = iota == idx2
        denom = jnp.maximum(m1 + m2, 1e-6)
        wte_ref[...] = (jnp.where(mask1, m1, 0.0) + jnp.where(mask2, m2, 0.0)) / denom
        g = jnp.dot(xb, s1_ref[...], preferred_element_type=jnp.float32)
        u = jnp.dot(xb, s3_ref[...], preferred_element_type=jnp.float32)
        g = jnp.minimum(g, LIMIT)
        u = jnp.clip(u, -LIMIT, LIMIT)
        h = (g * jax.nn.sigmoid(g)) * u
        out_ref[...] = jnp.dot(h, s2_ref[...], preferred_element_type=jnp.float32)

    iota = jax.lax.broadcasted_iota(jnp.int32, (TBLK, E), 1)
    w_col = jnp.sum(jnp.where(iota == e, wte_ref[...], 0.0), axis=1, keepdims=True)
    g = jnp.dot(xb, w1_ref[0], preferred_element_type=jnp.float32)
    u = jnp.dot(xb, w3_ref[0], preferred_element_type=jnp.float32)
    g = jnp.minimum(g, LIMIT)
    u = jnp.clip(u, -LIMIT, LIMIT)
    h = (g * jax.nn.sigmoid(g)) * u
    out_ref[...] += w_col * jnp.dot(h, w2_ref[0], preferred_element_type=jnp.float32)


@jax.jit
def _moe(flat, gate_w, W1, W2, W3, sw1, sw2, sw3):
    return pl.pallas_call(
        _moe_body,
        grid=(NT, E),
        in_specs=[
            pl.BlockSpec((TBLK, D), lambda t, e: (t, 0)),
            pl.BlockSpec((D, E), lambda t, e: (0, 0)),
            pl.BlockSpec((1, D, INTER), lambda t, e: (e, 0, 0)),
            pl.BlockSpec((1, INTER, D), lambda t, e: (e, 0, 0)),
            pl.BlockSpec((1, D, INTER), lambda t, e: (e, 0, 0)),
            pl.BlockSpec((D, INTER), lambda t, e: (0, 0)),
            pl.BlockSpec((INTER, D), lambda t, e: (0, 0)),
            pl.BlockSpec((D, INTER), lambda t, e: (0, 0)),
        ],
        out_specs=pl.BlockSpec((TBLK, D), lambda t, e: (t, 0)),
        out_shape=jax.ShapeDtypeStruct((T, D), jnp.float32),
        scratch_shapes=[pltpu.VMEM((TBLK, E), jnp.float32)],
    )(flat, gate_w, W1, W2, W3, sw1, sw2, sw3)


def kernel(x, input_ids, gate_w, W1, W2, W3, sw1, sw2, sw3):
    del input_ids
    flat = x.reshape(-1, D)
    out = _moe(flat, gate_w, W1, W2, W3, sw1, sw2, sw3)
    return out.reshape(x.shape)
